# baseline (speedup 1.0000x reference)
import functools
from itertools import combinations

import numpy as np
import jax
import jax.numpy as jnp
from jax import lax
from jax.experimental import pallas as pl
from jax.experimental.pallas import tpu as pltpu
from jax.experimental.pallas import tpu_sc as plsc

_B = 4096
_F = 26
_K = 16
_D0 = _F * _K
_BN_EPS = 1e-3
_FM_SCALE = 0.5 / float(np.sqrt(1.0 + _BN_EPS))

_NC = 2
_NS = 16
_NW = _NC * _NS
_SAMP_PER_W = _B // _NW
_NROW = _F
_IDX_PER_W = _SAMP_PER_W * _F
_XW_ROWS = 62500

_pairs = list(combinations(range(_F), 2))
_PCOLS = np.array([p[0] for p in _pairs], dtype=np.int32)
_PROWS = np.array([p[1] for p in _pairs], dtype=np.int32)


def _sc_gather_body(idx_in, xwrow_in, xv_hbm, xw2_hbm, xv_out, l_out,
                    idx_v, rowidx_v, rows_v, xwbuf_v, l_v, sem_v, sem_w):
    wid = lax.axis_index("s") * _NC + lax.axis_index("c")
    pltpu.sync_copy(idx_in.at[wid], idx_v)
    pltpu.sync_copy(xwrow_in.at[wid], rowidx_v)

    @pl.loop(0, _NROW)
    def _gather(j):
        sl = pl.ds(j * 128, 128)
        cv = pltpu.async_copy(xv_hbm.at[idx_v.at[sl]], rows_v.at[sl], sem_v)
        cw = pltpu.async_copy(xw2_hbm.at[rowidx_v.at[sl]], xwbuf_v.at[sl], sem_w)
        cv.wait()
        cw.wait()

    lanes16 = lax.iota(jnp.int32, 16)

    @pl.loop(0, _SAMP_PER_W // 16)
    def _linear(g):
        base = g * (16 * _F)

        def body(f, acc):
            pos16 = base + lanes16 * _F + f
            orig16 = plsc.load_gather(idx_v, [pos16])
            lane16 = lax.bitwise_and(orig16, 15)
            val16 = plsc.load_gather(xwbuf_v, [pos16, lane16])
            return acc + val16

        acc = pl.loop(0, _F, init_carry=jnp.zeros((16,), jnp.float32))(body)
        l_v[pl.ds(g * 16, 16)] = acc

    pltpu.sync_copy(rows_v, xv_out.at[wid])
    pltpu.sync_copy(l_v, l_out.at[wid])


def _sc_gather(idx2d, xwrow2d, xv_table, xw2_table):
    mesh = plsc.VectorSubcoreMesh(core_axis_name="c", subcore_axis_name="s")
    fn = pl.kernel(
        _sc_gather_body,
        out_type=[
            jax.ShapeDtypeStruct((_NW, _IDX_PER_W, _K), jnp.float32),
            jax.ShapeDtypeStruct((_NW, _SAMP_PER_W), jnp.float32),
        ],
        mesh=mesh,
        scratch_types=[
            pltpu.VMEM((_IDX_PER_W,), jnp.int32),
            pltpu.VMEM((_IDX_PER_W,), jnp.int32),
            pltpu.VMEM((_IDX_PER_W, _K), jnp.float32),
            pltpu.VMEM((_IDX_PER_W, _K), jnp.float32),
            pltpu.VMEM((_SAMP_PER_W,), jnp.float32),
            pltpu.SemaphoreType.DMA,
            pltpu.SemaphoreType.DMA,
        ],
        compiler_params=pltpu.CompilerParams(
            use_tc_tiling_on_sc=False, needs_layout_passes=False),
    )
    return fn(idx2d, xwrow2d, xv_table, xw2_table)


_BLK = 512


def _tc_body(xv_ref, l_ref, w0_ref, b0_ref, w1_ref, b1_ref, w2_ref, b2_ref,
             m_ref, logit_ref, sig_ref):
    x = xv_ref[...]
    h = jnp.dot(x, w0_ref[...], preferred_element_type=jnp.float32)
    h = jnp.maximum(h + b0_ref[...], 0.0)
    h = jnp.dot(h, w1_ref[...], preferred_element_type=jnp.float32)
    h = jnp.maximum(h + b1_ref[...], 0.0)
    hv = jnp.dot(h, w2_ref[...], preferred_element_type=jnp.float32)
    y = jnp.dot(x, m_ref[...], preferred_element_type=jnp.float32)
    fm = jnp.sum(x * y, axis=1, keepdims=True) * _FM_SCALE
    logit = l_ref[...] + fm + hv + b2_ref[...]
    logit_ref[...] = logit
    sig_ref[...] = jax.nn.sigmoid(logit)


def _tc_dense(xv_flat, l2d, W0, b0, W1, b1, W2, b2, M):
    nblk = _B // _BLK
    full = lambda s: pl.BlockSpec(s, lambda i: (0, 0))
    return pl.pallas_call(
        _tc_body,
        grid=(nblk,),
        in_specs=[
            pl.BlockSpec((_BLK, _D0), lambda i: (i, 0)),
            pl.BlockSpec((_BLK, 1), lambda i: (i, 0)),
            full(W0.shape), full((1, b0.shape[1])),
            full(W1.shape), full((1, b1.shape[1])),
            full(W2.shape), full((1, 1)),
            full(M.shape),
        ],
        out_specs=[
            pl.BlockSpec((_BLK, 1), lambda i: (i, 0)),
            pl.BlockSpec((_BLK, 1), lambda i: (i, 0)),
        ],
        out_shape=[
            jax.ShapeDtypeStruct((_B, 1), jnp.float32),
            jax.ShapeDtypeStruct((_B, 1), jnp.float32),
        ],
        compiler_params=pltpu.CompilerParams(
            dimension_semantics=("arbitrary",),
        ),
    )(xv_flat, l2d, W0, b0, W1, b1, W2, b2, M)


def kernel(inputs, xw_table, xv_table, W0, b0, W1, b1, W2, b2, edge_weights):
    idx = inputs.astype(jnp.int32)
    idx2d = idx.reshape(_NW, _IDX_PER_W)
    xwrow2d = lax.shift_right_logical(idx2d, 4)
    xw2_table = xw_table.reshape(_XW_ROWS, _K)

    xv_g, l_g = _sc_gather(idx2d, xwrow2d, xv_table, xw2_table)
    xv_flat = xv_g.reshape(_B, _D0)
    l2d = l_g.reshape(_B, 1)

    S = jnp.zeros((_F, _F), jnp.float32).at[_PROWS, _PCOLS].set(edge_weights)
    S = S + S.T
    M = jnp.kron(S, jnp.eye(_K, dtype=jnp.float32))

    logit2, sig2 = _tc_dense(
        xv_flat, l2d, W0, b0.reshape(1, -1), W1, b1.reshape(1, -1),
        W2, b2.reshape(1, 1), M)
    return logit2.reshape(_B), sig2.reshape(_B)


# speedup vs baseline: 1.0007x; 1.0007x over previous
import functools
from itertools import combinations

import numpy as np
import jax
import jax.numpy as jnp
from jax import lax
from jax.experimental import pallas as pl
from jax.experimental.pallas import tpu as pltpu
from jax.experimental.pallas import tpu_sc as plsc

_B = 4096
_F = 26
_K = 16
_D0 = _F * _K
_BN_EPS = 1e-3
_FM_SCALE = 0.5 / float(np.sqrt(1.0 + _BN_EPS))

_NC = 2
_NS = 16
_NW = _NC * _NS
_SAMP_PER_W = _B // _NW
_NROW = _F
_IDX_PER_W = _SAMP_PER_W * _F
_XW_ROWS = 62500

_pairs = list(combinations(range(_F), 2))
_PCOLS = np.array([p[0] for p in _pairs], dtype=np.int32)
_PROWS = np.array([p[1] for p in _pairs], dtype=np.int32)


def _sc_gather_body(idx_in, xwrow_in, xv_hbm, xw2_hbm, xv_out, l_out,
                    idx_v, rowidx_v, rows_v, xwbuf_v, l_v, sem_v, sem_w):
    wid = lax.axis_index("s") * _NC + lax.axis_index("c")
    pltpu.sync_copy(idx_in.at[wid], idx_v)
    pltpu.sync_copy(xwrow_in.at[wid], rowidx_v)

    @pl.loop(0, _NROW)
    def _gather(j):
        sl = pl.ds(j * 128, 128)
        pltpu.async_copy(xv_hbm.at[idx_v.at[sl]], rows_v.at[sl], sem_v)
        pltpu.async_copy(xw2_hbm.at[rowidx_v.at[sl]], xwbuf_v.at[sl], sem_w)

    pltpu.make_async_copy(xv_hbm.at[pl.ds(0, _IDX_PER_W)], rows_v, sem_v).wait()
    pltpu.make_async_copy(xw2_hbm.at[pl.ds(0, _IDX_PER_W)], xwbuf_v, sem_w).wait()

    lanes16 = lax.iota(jnp.int32, 16)

    @pl.loop(0, _SAMP_PER_W // 16)
    def _linear(g):
        base = g * (16 * _F)

        def body(f, acc):
            pos16 = base + lanes16 * _F + f
            orig16 = plsc.load_gather(idx_v, [pos16])
            lane16 = lax.bitwise_and(orig16, 15)
            val16 = plsc.load_gather(xwbuf_v, [pos16, lane16])
            return acc + val16

        acc = pl.loop(0, _F, init_carry=jnp.zeros((16,), jnp.float32))(body)
        l_v[pl.ds(g * 16, 16)] = acc

    pltpu.sync_copy(rows_v, xv_out.at[wid])
    pltpu.sync_copy(l_v, l_out.at[wid])


def _sc_gather(idx2d, xwrow2d, xv_table, xw2_table):
    mesh = plsc.VectorSubcoreMesh(core_axis_name="c", subcore_axis_name="s")
    fn = pl.kernel(
        _sc_gather_body,
        out_type=[
            jax.ShapeDtypeStruct((_NW, _IDX_PER_W, _K), jnp.float32),
            jax.ShapeDtypeStruct((_NW, _SAMP_PER_W), jnp.float32),
        ],
        mesh=mesh,
        scratch_types=[
            pltpu.VMEM((_IDX_PER_W,), jnp.int32),
            pltpu.VMEM((_IDX_PER_W,), jnp.int32),
            pltpu.VMEM((_IDX_PER_W, _K), jnp.float32),
            pltpu.VMEM((_IDX_PER_W, _K), jnp.float32),
            pltpu.VMEM((_SAMP_PER_W,), jnp.float32),
            pltpu.SemaphoreType.DMA,
            pltpu.SemaphoreType.DMA,
        ],
        compiler_params=pltpu.CompilerParams(
            use_tc_tiling_on_sc=False, needs_layout_passes=False),
    )
    return fn(idx2d, xwrow2d, xv_table, xw2_table)


_BLK = 512


def _tc_body(xv_ref, l_ref, w0_ref, b0_ref, w1_ref, b1_ref, w2_ref, b2_ref,
             m_ref, logit_ref, sig_ref):
    x = xv_ref[...]
    h = jnp.dot(x, w0_ref[...], preferred_element_type=jnp.float32)
    h = jnp.maximum(h + b0_ref[...], 0.0)
    h = jnp.dot(h, w1_ref[...], preferred_element_type=jnp.float32)
    h = jnp.maximum(h + b1_ref[...], 0.0)
    hv = jnp.dot(h, w2_ref[...], preferred_element_type=jnp.float32)
    y = jnp.dot(x, m_ref[...], preferred_element_type=jnp.float32)
    fm = jnp.sum(x * y, axis=1, keepdims=True) * _FM_SCALE
    logit = l_ref[...] + fm + hv + b2_ref[...]
    logit_ref[...] = logit
    sig_ref[...] = jax.nn.sigmoid(logit)


def _tc_dense(xv_flat, l2d, W0, b0, W1, b1, W2, b2, M):
    nblk = _B // _BLK
    full = lambda s: pl.BlockSpec(s, lambda i: (0, 0))
    return pl.pallas_call(
        _tc_body,
        grid=(nblk,),
        in_specs=[
            pl.BlockSpec((_BLK, _D0), lambda i: (i, 0)),
            pl.BlockSpec((_BLK, 1), lambda i: (i, 0)),
            full(W0.shape), full((1, b0.shape[1])),
            full(W1.shape), full((1, b1.shape[1])),
            full(W2.shape), full((1, 1)),
            full(M.shape),
        ],
        out_specs=[
            pl.BlockSpec((_BLK, 1), lambda i: (i, 0)),
            pl.BlockSpec((_BLK, 1), lambda i: (i, 0)),
        ],
        out_shape=[
            jax.ShapeDtypeStruct((_B, 1), jnp.float32),
            jax.ShapeDtypeStruct((_B, 1), jnp.float32),
        ],
        compiler_params=pltpu.CompilerParams(
            dimension_semantics=("arbitrary",),
        ),
    )(xv_flat, l2d, W0, b0, W1, b1, W2, b2, M)


def kernel(inputs, xw_table, xv_table, W0, b0, W1, b1, W2, b2, edge_weights):
    idx = inputs.astype(jnp.int32)
    idx2d = idx.reshape(_NW, _IDX_PER_W)
    xwrow2d = lax.shift_right_logical(idx2d, 4)
    xw2_table = xw_table.reshape(_XW_ROWS, _K)

    xv_g, l_g = _sc_gather(idx2d, xwrow2d, xv_table, xw2_table)
    xv_flat = xv_g.reshape(_B, _D0)
    l2d = l_g.reshape(_B, 1)

    S = jnp.zeros((_F, _F), jnp.float32).at[_PROWS, _PCOLS].set(edge_weights)
    S = S + S.T
    M = jnp.kron(S, jnp.eye(_K, dtype=jnp.float32))

    logit2, sig2 = _tc_dense(
        xv_flat, l2d, W0, b0.reshape(1, -1), W1, b1.reshape(1, -1),
        W2, b2.reshape(1, 1), M)
    return logit2.reshape(_B), sig2.reshape(_B)


# speedup vs baseline: 1.0276x; 1.0269x over previous
import functools
from itertools import combinations

import numpy as np
import jax
import jax.numpy as jnp
from jax import lax
from jax.experimental import pallas as pl
from jax.experimental.pallas import tpu as pltpu
from jax.experimental.pallas import tpu_sc as plsc

_B = 4096
_F = 26
_K = 16
_D0 = _F * _K
_BN_EPS = 1e-3
_FM_SCALE = 0.5 / float(np.sqrt(1.0 + _BN_EPS))

_NC = 2
_NS = 16
_NW = _NC * _NS
_SAMP_PER_W = _B // _NW
_NROW = _F
_IDX_PER_W = _SAMP_PER_W * _F
_XW_ROWS = 62500

_pairs = list(combinations(range(_F), 2))
_NPAIRS = len(_pairs)
_ONEHOT_SYM = np.zeros((_NPAIRS, _F * _F), dtype=np.float32)
for _p, (_c, _r) in enumerate(_pairs):
    _ONEHOT_SYM[_p, _r * _F + _c] = _FM_SCALE
    _ONEHOT_SYM[_p, _c * _F + _r] = _FM_SCALE
_EYE_K = np.eye(_K, dtype=np.float32)


def _sc_gather_body(idx_in, xwrow_in, xv_hbm, xw2_hbm, xv_out, l_out,
                    idx_v, rowidx_v, rows_v, xwbuf_v, l_v, sem_v, sem_w):
    wid = lax.axis_index("s") * _NC + lax.axis_index("c")
    pltpu.sync_copy(idx_in.at[wid], idx_v)
    pltpu.sync_copy(xwrow_in.at[wid], rowidx_v)

    @pl.loop(0, _NROW)
    def _gather(j):
        sl = pl.ds(j * 128, 128)
        pltpu.async_copy(xv_hbm.at[idx_v.at[sl]], rows_v.at[sl], sem_v)
        pltpu.async_copy(xw2_hbm.at[rowidx_v.at[sl]], xwbuf_v.at[sl], sem_w)

    pltpu.make_async_copy(xv_hbm.at[pl.ds(0, _IDX_PER_W)], rows_v, sem_v).wait()
    pltpu.make_async_copy(xw2_hbm.at[pl.ds(0, _IDX_PER_W)], xwbuf_v, sem_w).wait()

    lanes16 = lax.iota(jnp.int32, 16)

    @pl.loop(0, _SAMP_PER_W // 16)
    def _linear(g):
        base = g * (16 * _F)

        def body(f, acc):
            pos16 = base + lanes16 * _F + f
            orig16 = plsc.load_gather(idx_v, [pos16])
            lane16 = lax.bitwise_and(orig16, 15)
            val16 = plsc.load_gather(xwbuf_v, [pos16, lane16])
            return acc + val16

        acc = pl.loop(0, _F, init_carry=jnp.zeros((16,), jnp.float32))(body)
        l_v[pl.ds(g * 16, 16)] = acc

    pltpu.sync_copy(rows_v, xv_out.at[wid])
    pltpu.sync_copy(l_v, l_out.at[wid])


def _sc_gather(idx2d, xwrow2d, xv_table, xw2_table):
    mesh = plsc.VectorSubcoreMesh(core_axis_name="c", subcore_axis_name="s")
    fn = pl.kernel(
        _sc_gather_body,
        out_type=[
            jax.ShapeDtypeStruct((_NW, _IDX_PER_W, _K), jnp.float32),
            jax.ShapeDtypeStruct((_NW, _SAMP_PER_W), jnp.float32),
        ],
        mesh=mesh,
        scratch_types=[
            pltpu.VMEM((_IDX_PER_W,), jnp.int32),
            pltpu.VMEM((_IDX_PER_W,), jnp.int32),
            pltpu.VMEM((_IDX_PER_W, _K), jnp.float32),
            pltpu.VMEM((_IDX_PER_W, _K), jnp.float32),
            pltpu.VMEM((_SAMP_PER_W,), jnp.float32),
            pltpu.SemaphoreType.DMA,
            pltpu.SemaphoreType.DMA,
        ],
        compiler_params=pltpu.CompilerParams(
            use_tc_tiling_on_sc=False, needs_layout_passes=False),
    )
    return fn(idx2d, xwrow2d, xv_table, xw2_table)


_BLK = 512


def _tc_body(xv_ref, l_ref, w0_ref, b0_ref, w1_ref, b1_ref, w2_ref, b2_ref,
             m_ref, logit_ref, sig_ref):
    x = xv_ref[...]
    h = jnp.dot(x, w0_ref[...], preferred_element_type=jnp.float32)
    h = jnp.maximum(h + b0_ref[...], 0.0)
    h = jnp.dot(h, w1_ref[...], preferred_element_type=jnp.float32)
    h = jnp.maximum(h + b1_ref[...], 0.0)
    hv = jnp.dot(h, w2_ref[...], preferred_element_type=jnp.float32)
    y = jnp.dot(x, m_ref[...], preferred_element_type=jnp.float32)
    fm = jnp.sum(x * y, axis=1, keepdims=True)
    logit = l_ref[...] + fm + hv + b2_ref[...]
    logit_ref[...] = logit
    sig_ref[...] = jax.nn.sigmoid(logit)


def _tc_dense(xv_flat, l2d, W0, b0, W1, b1, W2, b2, M):
    nblk = _B // _BLK
    full = lambda s: pl.BlockSpec(s, lambda i: (0, 0))
    return pl.pallas_call(
        _tc_body,
        grid=(nblk,),
        in_specs=[
            pl.BlockSpec((_BLK, _D0), lambda i: (i, 0)),
            pl.BlockSpec((_BLK, 1), lambda i: (i, 0)),
            full(W0.shape), full((1, b0.shape[1])),
            full(W1.shape), full((1, b1.shape[1])),
            full(W2.shape), full((1, 1)),
            full(M.shape),
        ],
        out_specs=[
            pl.BlockSpec((_BLK, 1), lambda i: (i, 0)),
            pl.BlockSpec((_BLK, 1), lambda i: (i, 0)),
        ],
        out_shape=[
            jax.ShapeDtypeStruct((_B, 1), jnp.float32),
            jax.ShapeDtypeStruct((_B, 1), jnp.float32),
        ],
        compiler_params=pltpu.CompilerParams(
            dimension_semantics=("arbitrary",),
        ),
    )(xv_flat, l2d, W0, b0, W1, b1, W2, b2, M)


def kernel(inputs, xw_table, xv_table, W0, b0, W1, b1, W2, b2, edge_weights):
    idx = inputs.astype(jnp.int32)
    idx2d = idx.reshape(_NW, _IDX_PER_W)
    xwrow2d = lax.shift_right_logical(idx2d, 4)
    xw2_table = xw_table.reshape(_XW_ROWS, _K)

    xv_g, l_g = _sc_gather(idx2d, xwrow2d, xv_table, xw2_table)
    xv_flat = xv_g.reshape(_B, _D0)
    l2d = l_g.reshape(_B, 1)

    S = (edge_weights @ _ONEHOT_SYM).reshape(_F, _F)
    M = (S[:, None, :, None] * _EYE_K[None, :, None, :]).reshape(_D0, _D0)

    logit2, sig2 = _tc_dense(
        xv_flat, l2d, W0, b0.reshape(1, -1), W1, b1.reshape(1, -1),
        W2, b2.reshape(1, 1), M)
    return logit2.reshape(_B), sig2.reshape(_B)


# speedup vs baseline: 1.0300x; 1.0023x over previous
import functools
from itertools import combinations

import numpy as np
import jax
import jax.numpy as jnp
from jax import lax
from jax.experimental import pallas as pl
from jax.experimental.pallas import tpu as pltpu
from jax.experimental.pallas import tpu_sc as plsc

_B = 4096
_F = 26
_K = 16
_D0 = _F * _K
_BN_EPS = 1e-3
_FM_SCALE = 0.5 / float(np.sqrt(1.0 + _BN_EPS))

_NC = 2
_NS = 16
_NW = _NC * _NS
_SAMP_PER_W = _B // _NW
_NROW = _F
_IDX_PER_W = _SAMP_PER_W * _F
_XW_ROWS = 62500

_pairs = list(combinations(range(_F), 2))
_NPAIRS = len(_pairs)
_ONEHOT_SYM = np.zeros((_NPAIRS, _F * _F), dtype=np.float32)
for _p, (_c, _r) in enumerate(_pairs):
    _ONEHOT_SYM[_p, _r * _F + _c] = _FM_SCALE
    _ONEHOT_SYM[_p, _c * _F + _r] = _FM_SCALE
_EYE_K = np.eye(_K, dtype=np.float32)


def _sc_gather_body(idx_in, xwrow_in, xv_hbm, xw2_hbm, xv_out, l_out,
                    idx_v, rowidx_v, rows_v, xwbuf_v, l_v, sem_v, sem_w):
    wid = lax.axis_index("s") * _NC + lax.axis_index("c")
    pltpu.sync_copy(idx_in.at[wid], idx_v)
    pltpu.sync_copy(xwrow_in.at[wid], rowidx_v)

    @pl.loop(0, _NROW)
    def _gather(j):
        sl = pl.ds(j * 128, 128)
        pltpu.async_copy(xv_hbm.at[idx_v.at[sl]], rows_v.at[sl], sem_v)
        pltpu.async_copy(xw2_hbm.at[rowidx_v.at[sl]], xwbuf_v.at[sl], sem_w)

    pltpu.make_async_copy(xv_hbm.at[pl.ds(0, _IDX_PER_W)], rows_v, sem_v).wait()
    pltpu.make_async_copy(xw2_hbm.at[pl.ds(0, _IDX_PER_W)], xwbuf_v, sem_w).wait()

    lanes16 = lax.iota(jnp.int32, 16)

    @pl.loop(0, _SAMP_PER_W // 16)
    def _linear(g):
        base = g * (16 * _F)

        def body(f, acc):
            pos16 = base + lanes16 * _F + f
            orig16 = plsc.load_gather(idx_v, [pos16])
            lane16 = lax.bitwise_and(orig16, 15)
            val16 = plsc.load_gather(xwbuf_v, [pos16, lane16])
            return acc + val16

        acc = pl.loop(0, _F, init_carry=jnp.zeros((16,), jnp.float32))(body)
        l_v[pl.ds(g * 16, 16)] = acc

    pltpu.sync_copy(rows_v, xv_out.at[wid])
    pltpu.sync_copy(l_v, l_out.at[wid])


def _sc_gather(idx2d, xwrow2d, xv_table, xw2_table):
    mesh = plsc.VectorSubcoreMesh(core_axis_name="c", subcore_axis_name="s")
    fn = pl.kernel(
        _sc_gather_body,
        out_type=[
            jax.ShapeDtypeStruct((_NW, _IDX_PER_W, _K), jnp.float32),
            jax.ShapeDtypeStruct((_NW, _SAMP_PER_W), jnp.float32),
        ],
        mesh=mesh,
        scratch_types=[
            pltpu.VMEM((_IDX_PER_W,), jnp.int32),
            pltpu.VMEM((_IDX_PER_W,), jnp.int32),
            pltpu.VMEM((_IDX_PER_W, _K), jnp.float32),
            pltpu.VMEM((_IDX_PER_W, _K), jnp.float32),
            pltpu.VMEM((_SAMP_PER_W,), jnp.float32),
            pltpu.SemaphoreType.DMA,
            pltpu.SemaphoreType.DMA,
        ],
        compiler_params=pltpu.CompilerParams(
            use_tc_tiling_on_sc=False, needs_layout_passes=False),
    )
    return fn(idx2d, xwrow2d, xv_table, xw2_table)


_TVOC = 16384
_TROWS = _TVOC * _K // 128
_VROWS = 1000000 * _K // 128


def _tr_body(x_ref, o_ref):
    x = x_ref[...]
    x3 = x.reshape(_K, _TROWS, 8)
    o_ref[...] = x3.transpose(1, 2, 0).reshape(_TROWS, 128)


def _tc_relayout(xt):
    grid = (pl.cdiv(1000000, _TVOC),)
    return pl.pallas_call(
        _tr_body,
        grid=grid,
        in_specs=[pl.BlockSpec((_K, _TVOC), lambda i: (0, i))],
        out_specs=pl.BlockSpec((_TROWS, 128), lambda i: (i, 0)),
        out_shape=jax.ShapeDtypeStruct((_VROWS, 128), jnp.float32),
        compiler_params=pltpu.CompilerParams(
            dimension_semantics=("arbitrary",),
        ),
    )(xt)


_BLK = 512


def _tc_body(xv_ref, l_ref, w0_ref, b0_ref, w1_ref, b1_ref, w2_ref, b2_ref,
             m_ref, logit_ref, sig_ref):
    x = xv_ref[...]
    h = jnp.dot(x, w0_ref[...], preferred_element_type=jnp.float32)
    h = jnp.maximum(h + b0_ref[...], 0.0)
    h = jnp.dot(h, w1_ref[...], preferred_element_type=jnp.float32)
    h = jnp.maximum(h + b1_ref[...], 0.0)
    hv = jnp.dot(h, w2_ref[...], preferred_element_type=jnp.float32)
    y = jnp.dot(x, m_ref[...], preferred_element_type=jnp.float32)
    fm = jnp.sum(x * y, axis=1, keepdims=True)
    logit = l_ref[...] + fm + hv + b2_ref[...]
    logit_ref[...] = logit
    sig_ref[...] = jax.nn.sigmoid(logit)


def _tc_dense(xv_flat, l2d, W0, b0, W1, b1, W2, b2, M):
    nblk = _B // _BLK
    full = lambda s: pl.BlockSpec(s, lambda i: (0, 0))
    return pl.pallas_call(
        _tc_body,
        grid=(nblk,),
        in_specs=[
            pl.BlockSpec((_BLK, _D0), lambda i: (i, 0)),
            pl.BlockSpec((_BLK, 1), lambda i: (i, 0)),
            full(W0.shape), full((1, b0.shape[1])),
            full(W1.shape), full((1, b1.shape[1])),
            full(W2.shape), full((1, 1)),
            full(M.shape),
        ],
        out_specs=[
            pl.BlockSpec((_BLK, 1), lambda i: (i, 0)),
            pl.BlockSpec((_BLK, 1), lambda i: (i, 0)),
        ],
        out_shape=[
            jax.ShapeDtypeStruct((_B, 1), jnp.float32),
            jax.ShapeDtypeStruct((_B, 1), jnp.float32),
        ],
        compiler_params=pltpu.CompilerParams(
            dimension_semantics=("arbitrary",),
        ),
    )(xv_flat, l2d, W0, b0, W1, b1, W2, b2, M)


def kernel(inputs, xw_table, xv_table, W0, b0, W1, b1, W2, b2, edge_weights):
    idx = inputs.astype(jnp.int32)
    idx2d = idx.reshape(_NW, _IDX_PER_W)
    xwrow2d = lax.shift_right_logical(idx2d, 4)
    xw2_table = xw_table.reshape(_XW_ROWS, _K)

    xv_lin = lax.optimization_barrier(xv_table.reshape(_VROWS, 128))
    xv_rm = xv_lin.reshape(1000000, _K)

    xv_g, l_g = _sc_gather(idx2d, xwrow2d, xv_rm, xw2_table)
    xv_flat = xv_g.reshape(_B, _D0)
    l2d = l_g.reshape(_B, 1)

    S = (edge_weights @ _ONEHOT_SYM).reshape(_F, _F)
    M = (S[:, None, :, None] * _EYE_K[None, :, None, :]).reshape(_D0, _D0)

    logit2, sig2 = _tc_dense(
        xv_flat, l2d, W0, b0.reshape(1, -1), W1, b1.reshape(1, -1),
        W2, b2.reshape(1, 1), M)
    return logit2.reshape(_B), sig2.reshape(_B)
